# Initial kernel scaffold; baseline (speedup 1.0000x reference)
import functools

import jax
import jax.numpy as jnp
from jax import lax
from jax.experimental import pallas as pl
from jax.experimental.pallas import tpu as pltpu
import jax.experimental.pallas.tpu_sc as plsc

N_NODES = 10000
N_EDGES = 640000
N_SOLV = 2560
NUM_GRAPHS = 128
D_ATOM = 40
EMB = 64
NUM_LAYER = 3

NC = 2
NS = 16
NW = NC * NS
EB = 128
STEPS = 158
E_PAD = NW * STEPS * EB
N_PAD = 10240
R_T = N_PAD // NS

_HIGH = jax.lax.Precision.HIGHEST


def _edge_agg_body(h_hbm, src_hbm, dst_hbm, z_hbm, out_hbm,
                   src_v, dst_v, rows_v, agg_sh, gsem):
    c = lax.axis_index("c")
    s = lax.axis_index("s")
    wid = c * NS + s
    pltpu.sync_copy(z_hbm, agg_sh.at[pl.ds(s * R_T, R_T)])
    pltpu.sync_copy(src_hbm.at[wid], src_v)
    pltpu.sync_copy(dst_hbm.at[wid], dst_v)
    plsc.subcore_barrier()

    def step(j, carry):
        pltpu.async_copy(h_hbm.at[src_v.at[j]], rows_v, gsem).wait()
        pltpu.sync_copy(rows_v, agg_sh.at[dst_v.at[j]], add=True)
        return carry

    lax.fori_loop(0, STEPS, step, 0)
    plsc.subcore_barrier()
    base = c * N_PAD + s * R_T
    pltpu.sync_copy(agg_sh.at[pl.ds(s * R_T, R_T)], out_hbm.at[pl.ds(base, R_T)])


def _edge_agg(h, src_r, dst_r, z):
    mesh = plsc.VectorSubcoreMesh(core_axis_name="c", subcore_axis_name="s")
    return pl.kernel(
        _edge_agg_body,
        out_type=jax.ShapeDtypeStruct((NC * N_PAD, EMB), jnp.float32),
        mesh=mesh,
        scratch_types=[
            pltpu.VMEM((STEPS, EB), jnp.int32),
            pltpu.VMEM((STEPS, EB), jnp.int32),
            pltpu.VMEM((EB, EMB), jnp.float32),
            pltpu.VMEM_SHARED((N_PAD, EMB), jnp.float32),
            pltpu.SemaphoreType.DMA,
        ],
    )(h, src_r, dst_r, z)


def _embed_body(x_ref, w_ref, b_ref, o_ref):
    o_ref[...] = jax.nn.relu(
        jnp.dot(x_ref[...], w_ref[...], precision=_HIGH,
                preferred_element_type=jnp.float32) + b_ref[...])


def _embed(x, w, b):
    return pl.pallas_call(
        _embed_body,
        out_shape=jax.ShapeDtypeStruct((N_NODES, EMB), jnp.float32),
    )(x, w, b)


def _layer_body(p0_ref, p1_ref, w_ref, b_ref, g_ref, be_ref, o_ref, *, relu):
    agg = p0_ref[...] + p1_ref[...]
    hl = jnp.dot(agg, w_ref[...], precision=_HIGH,
                 preferred_element_type=jnp.float32) + b_ref[...]
    mean = jnp.mean(hl, axis=0, keepdims=True)
    cen = hl - mean
    var = jnp.mean(cen * cen, axis=0, keepdims=True)
    hl = cen * jax.lax.rsqrt(var + 1e-5) * g_ref[...] + be_ref[...]
    if relu:
        hl = jax.nn.relu(hl)
    o_ref[...] = hl


def _layer(p, w, b, g, be, relu):
    p0 = p[:N_NODES]
    p1 = p[N_PAD:N_PAD + N_NODES]
    return pl.pallas_call(
        functools.partial(_layer_body, relu=relu),
        out_shape=jax.ShapeDtypeStruct((N_NODES, EMB), jnp.float32),
    )(p0, p1, w, b, g, be)


def _solvent_body(x_ref, w1_ref, b1_ref, w2_ref, b2_ref, o_ref):
    h = jax.nn.relu(
        jnp.dot(x_ref[...], w1_ref[...], precision=_HIGH,
                preferred_element_type=jnp.float32) + b1_ref[...])
    o_ref[...] = jnp.dot(h, w2_ref[...], precision=_HIGH,
                         preferred_element_type=jnp.float32) + b2_ref[...]


def _solvent(x, w1, b1, w2, b2):
    return pl.pallas_call(
        _solvent_body,
        out_shape=jax.ShapeDtypeStruct((N_SOLV, EMB), jnp.float32),
    )(x, w1, b1, w2, b2)


SB = 400
NSB = N_NODES // SB


def _interact_body(srep_ref, vrep_ref, sb_ref, vb_ref,
                   wo0_ref, bo0_ref, wo1_ref, bo1_ref, wo2_ref, bo2_ref,
                   wl_ref, bl_ref, o_ref,
                   sfsum, scnt, vprime):
    i = pl.program_id(0)
    vrep = vrep_ref[...]

    @pl.when(i == 0)
    def _init():
        sfsum[...] = jnp.zeros_like(sfsum)
        scnt[...] = jnp.zeros_like(scnt)
        vprime[...] = jnp.zeros_like(vprime)

    rep = srep_ref[...]
    sb = sb_ref[...]
    mask = (sb == vb_ref[...].reshape(1, N_SOLV)).astype(jnp.float32)
    imap = lax.dot_general(rep, vrep, (((1,), (1,)), ((), ())),
                           precision=_HIGH,
                           preferred_element_type=jnp.float32) * mask
    sp = jnp.dot(imap, vrep, precision=_HIGH,
                 preferred_element_type=jnp.float32)
    sf = rep + sp
    gids = lax.broadcasted_iota(jnp.float32, (NUM_GRAPHS, 1), 0)
    p_s = (gids == sb.reshape(1, SB)).astype(jnp.float32)
    sfsum[...] += jnp.dot(p_s, sf, precision=_HIGH,
                          preferred_element_type=jnp.float32)
    scnt[...] += jnp.sum(p_s, axis=1, keepdims=True)
    vprime[...] += lax.dot_general(imap, rep, (((0,), (0,)), ((), ())),
                                   precision=_HIGH,
                                   preferred_element_type=jnp.float32)

    @pl.when(i == NSB - 1)
    def _final():
        vf = vrep + vprime[...]
        p_v = (gids == vb_ref[...].reshape(1, N_SOLV)).astype(jnp.float32)
        vsum = jnp.dot(p_v, vf, precision=_HIGH,
                       preferred_element_type=jnp.float32)
        vcnt = jnp.sum(p_v, axis=1, keepdims=True)
        pooled_s = sfsum[...] / jnp.maximum(scnt[...], 1.0)
        pooled_v = vsum / jnp.maximum(vcnt, 1.0)
        final = jnp.concatenate([pooled_s, pooled_v], axis=1)
        h1 = jax.nn.relu(jnp.dot(final, wo0_ref[...], precision=_HIGH,
                                 preferred_element_type=jnp.float32) + bo0_ref[...])
        h2 = jax.nn.relu(jnp.dot(h1, wo1_ref[...], precision=_HIGH,
                                 preferred_element_type=jnp.float32) + bo1_ref[...])
        h3 = jax.nn.relu(jnp.dot(h2, wo2_ref[...], precision=_HIGH,
                                 preferred_element_type=jnp.float32) + bo2_ref[...])
        o_ref[...] = jnp.dot(h3, wl_ref[...], precision=_HIGH,
                             preferred_element_type=jnp.float32) + bl_ref[...]


def _interact(srep, vrep, sb_f, vb_f, wo0, bo0, wo1, bo1, wo2, bo2, wl, bl):
    const = lambda i: (0, 0)
    return pl.pallas_call(
        _interact_body,
        grid=(NSB,),
        in_specs=[
            pl.BlockSpec((SB, EMB), lambda i: (i, 0)),
            pl.BlockSpec((N_SOLV, EMB), const),
            pl.BlockSpec((SB, 1), lambda i: (i, 0)),
            pl.BlockSpec((N_SOLV, 1), const),
            pl.BlockSpec((2 * EMB, EMB), const),
            pl.BlockSpec((1, EMB), const),
            pl.BlockSpec((EMB, EMB // 2), const),
            pl.BlockSpec((1, EMB // 2), const),
            pl.BlockSpec((EMB // 2, EMB // 4), const),
            pl.BlockSpec((1, EMB // 4), const),
            pl.BlockSpec((EMB // 4, 1), const),
            pl.BlockSpec((1, 1), const),
        ],
        out_specs=pl.BlockSpec((NUM_GRAPHS, 1), const),
        out_shape=jax.ShapeDtypeStruct((NUM_GRAPHS, 1), jnp.float32),
        scratch_shapes=[
            pltpu.VMEM((NUM_GRAPHS, EMB), jnp.float32),
            pltpu.VMEM((NUM_GRAPHS, 1), jnp.float32),
            pltpu.VMEM((N_SOLV, EMB), jnp.float32),
        ],
    )(srep, vrep, sb_f, vb_f, wo0, bo0, wo1, bo1, wo2, bo2, wl, bl)


def kernel(solute_x, solute_edge_index, solute_batch, solvent_x, solvent_batch,
           W_embed, b_embed, W_gcn, b_gcn, bn_gamma, bn_beta,
           Ws1, bs1, Ws2, bs2, Wo0, bo0, Wo1, bo1, Wo2, bo2, Wlast, blast):
    src = solute_edge_index[0]
    dst = solute_edge_index[1]
    pad = E_PAD - N_EDGES
    src_r = jnp.concatenate(
        [src, jnp.zeros((pad,), jnp.int32)]).reshape(NW, STEPS, EB)
    dst_r = jnp.concatenate(
        [dst, jnp.full((pad,), N_NODES, jnp.int32)]).reshape(NW, STEPS, EB)
    z = jnp.zeros((R_T, EMB), jnp.float32)

    h = _embed(solute_x, W_embed, b_embed.reshape(1, EMB))
    for l in range(NUM_LAYER):
        p = _edge_agg(h, src_r, dst_r, z)
        h = _layer(p, W_gcn[l], b_gcn[l].reshape(1, EMB),
                   bn_gamma[l].reshape(1, EMB), bn_beta[l].reshape(1, EMB),
                   relu=(l < NUM_LAYER - 1))

    vrep = _solvent(solvent_x, Ws1, bs1.reshape(1, EMB), Ws2, bs2.reshape(1, EMB))
    out = _interact(h, vrep,
                    solute_batch.astype(jnp.float32).reshape(N_NODES, 1),
                    solvent_batch.astype(jnp.float32).reshape(N_SOLV, 1),
                    Wo0, bo0.reshape(1, EMB),
                    Wo1, bo1.reshape(1, EMB // 2),
                    Wo2, bo2.reshape(1, EMB // 4),
                    Wlast, blast.reshape(1, 1))
    return out


# speedup vs baseline: 8.1118x; 8.1118x over previous
import functools

import jax
import jax.numpy as jnp
from jax import lax
from jax.experimental import pallas as pl
from jax.experimental.pallas import tpu as pltpu
import jax.experimental.pallas.tpu_sc as plsc

N_NODES = 10000
N_EDGES = 640000
N_SOLV = 2560
NUM_GRAPHS = 128
D_ATOM = 40
EMB = 64
NUM_LAYER = 3

NC = 2
NS = 16
NW = NC * NS
EB = 128
STEPS = 158
E_PAD = NW * STEPS * EB
N_PAD = 10240
R_T = N_PAD // NS

_HIGH = jax.lax.Precision.DEFAULT


def _edge_agg_body(h_hbm, src_hbm, dst_hbm, z_hbm, out_hbm,
                   src_v, dst_v, rows_v, agg_sh, gsem):
    c = lax.axis_index("c")
    s = lax.axis_index("s")
    wid = c * NS + s
    pltpu.sync_copy(z_hbm, agg_sh.at[pl.ds(s * R_T, R_T)])
    pltpu.sync_copy(src_hbm.at[wid], src_v)
    pltpu.sync_copy(dst_hbm.at[wid], dst_v)
    plsc.subcore_barrier()

    def step(j, carry):
        pltpu.async_copy(h_hbm.at[src_v.at[j]], rows_v, gsem).wait()
        pltpu.sync_copy(rows_v, agg_sh.at[dst_v.at[j]], add=True)
        return carry

    lax.fori_loop(0, STEPS, step, 0)
    plsc.subcore_barrier()
    base = c * N_PAD + s * R_T
    pltpu.sync_copy(agg_sh.at[pl.ds(s * R_T, R_T)], out_hbm.at[pl.ds(base, R_T)])


def _edge_agg(h, src_r, dst_r, z):
    mesh = plsc.VectorSubcoreMesh(core_axis_name="c", subcore_axis_name="s",
                                  num_cores=NC, num_subcores=NS)
    return pl.kernel(
        _edge_agg_body,
        out_type=jax.ShapeDtypeStruct((NC * N_PAD, EMB), jnp.float32),
        mesh=mesh,
        scratch_types=[
            pltpu.VMEM((STEPS, EB), jnp.int32),
            pltpu.VMEM((STEPS, EB), jnp.int32),
            pltpu.VMEM((EB, EMB), jnp.float32),
            pltpu.VMEM_SHARED((N_PAD, EMB), jnp.float32),
            pltpu.SemaphoreType.DMA,
        ],
        compiler_params=pltpu.CompilerParams(use_tc_tiling_on_sc=False),
    )(h, src_r, dst_r, z)


def _embed_body(x_ref, w_ref, b_ref, o_ref):
    o_ref[...] = jax.nn.relu(
        jnp.dot(x_ref[...], w_ref[...], precision=_HIGH,
                preferred_element_type=jnp.float32) + b_ref[...])


def _embed(x, w, b):
    return pl.pallas_call(
        _embed_body,
        out_shape=jax.ShapeDtypeStruct((N_NODES, EMB), jnp.float32),
    )(x, w, b)


def _layer_body(p0_ref, p1_ref, w_ref, b_ref, g_ref, be_ref, o_ref, *, relu):
    agg = p0_ref[...] + p1_ref[...]
    hl = jnp.dot(agg, w_ref[...], precision=_HIGH,
                 preferred_element_type=jnp.float32) + b_ref[...]
    mean = jnp.mean(hl, axis=0, keepdims=True)
    cen = hl - mean
    var = jnp.mean(cen * cen, axis=0, keepdims=True)
    hl = cen * jax.lax.rsqrt(var + 1e-5) * g_ref[...] + be_ref[...]
    if relu:
        hl = jax.nn.relu(hl)
    o_ref[...] = hl


def _layer(p, w, b, g, be, relu):
    p0 = p[:N_NODES]
    p1 = p[N_PAD:N_PAD + N_NODES]
    return pl.pallas_call(
        functools.partial(_layer_body, relu=relu),
        out_shape=jax.ShapeDtypeStruct((N_NODES, EMB), jnp.float32),
    )(p0, p1, w, b, g, be)


def _solvent_body(x_ref, w1_ref, b1_ref, w2_ref, b2_ref, o_ref):
    h = jax.nn.relu(
        jnp.dot(x_ref[...], w1_ref[...], precision=_HIGH,
                preferred_element_type=jnp.float32) + b1_ref[...])
    o_ref[...] = jnp.dot(h, w2_ref[...], precision=_HIGH,
                         preferred_element_type=jnp.float32) + b2_ref[...]


def _solvent(x, w1, b1, w2, b2):
    return pl.pallas_call(
        _solvent_body,
        out_shape=jax.ShapeDtypeStruct((N_SOLV, EMB), jnp.float32),
    )(x, w1, b1, w2, b2)


SB = 400
NSB = N_NODES // SB


def _interact_body(srep_ref, vrep_ref, sb_ref, vb_ref,
                   wo0_ref, bo0_ref, wo1_ref, bo1_ref, wo2_ref, bo2_ref,
                   wl_ref, bl_ref, o_ref,
                   sfsum, scnt, vprime):
    i = pl.program_id(0)
    vrep = vrep_ref[...]

    @pl.when(i == 0)
    def _init():
        sfsum[...] = jnp.zeros_like(sfsum)
        scnt[...] = jnp.zeros_like(scnt)
        vprime[...] = jnp.zeros_like(vprime)

    rep = srep_ref[...]
    sb = sb_ref[...]
    mask = (sb == vb_ref[...].reshape(1, N_SOLV)).astype(jnp.float32)
    imap = lax.dot_general(rep, vrep, (((1,), (1,)), ((), ())),
                           precision=_HIGH,
                           preferred_element_type=jnp.float32) * mask
    sp = jnp.dot(imap, vrep, precision=_HIGH,
                 preferred_element_type=jnp.float32)
    sf = rep + sp
    gids = lax.broadcasted_iota(jnp.int32, (NUM_GRAPHS, 1), 0).astype(jnp.float32)
    p_s = (gids == sb.reshape(1, SB)).astype(jnp.float32)
    sfsum[...] += jnp.dot(p_s, sf, precision=_HIGH,
                          preferred_element_type=jnp.float32)
    scnt[...] += jnp.sum(p_s, axis=1, keepdims=True)
    vprime[...] += lax.dot_general(imap, rep, (((0,), (0,)), ((), ())),
                                   precision=_HIGH,
                                   preferred_element_type=jnp.float32)

    @pl.when(i == NSB - 1)
    def _final():
        vf = vrep + vprime[...]
        p_v = (gids == vb_ref[...].reshape(1, N_SOLV)).astype(jnp.float32)
        vsum = jnp.dot(p_v, vf, precision=_HIGH,
                       preferred_element_type=jnp.float32)
        vcnt = jnp.sum(p_v, axis=1, keepdims=True)
        pooled_s = sfsum[...] / jnp.maximum(scnt[...], 1.0)
        pooled_v = vsum / jnp.maximum(vcnt, 1.0)
        final = jnp.concatenate([pooled_s, pooled_v], axis=1)
        h1 = jax.nn.relu(jnp.dot(final, wo0_ref[...], precision=_HIGH,
                                 preferred_element_type=jnp.float32) + bo0_ref[...])
        h2 = jax.nn.relu(jnp.dot(h1, wo1_ref[...], precision=_HIGH,
                                 preferred_element_type=jnp.float32) + bo1_ref[...])
        h3 = jax.nn.relu(jnp.dot(h2, wo2_ref[...], precision=_HIGH,
                                 preferred_element_type=jnp.float32) + bo2_ref[...])
        o_ref[...] = jnp.dot(h3, wl_ref[...], precision=_HIGH,
                             preferred_element_type=jnp.float32) + bl_ref[...]


def _interact(srep, vrep, sb_f, vb_f, wo0, bo0, wo1, bo1, wo2, bo2, wl, bl):
    const = lambda i: (0, 0)
    return pl.pallas_call(
        _interact_body,
        grid=(NSB,),
        in_specs=[
            pl.BlockSpec((SB, EMB), lambda i: (i, 0)),
            pl.BlockSpec((N_SOLV, EMB), const),
            pl.BlockSpec((SB, 1), lambda i: (i, 0)),
            pl.BlockSpec((N_SOLV, 1), const),
            pl.BlockSpec((2 * EMB, EMB), const),
            pl.BlockSpec((1, EMB), const),
            pl.BlockSpec((EMB, EMB // 2), const),
            pl.BlockSpec((1, EMB // 2), const),
            pl.BlockSpec((EMB // 2, EMB // 4), const),
            pl.BlockSpec((1, EMB // 4), const),
            pl.BlockSpec((EMB // 4, 1), const),
            pl.BlockSpec((1, 1), const),
        ],
        out_specs=pl.BlockSpec((NUM_GRAPHS, 1), const),
        out_shape=jax.ShapeDtypeStruct((NUM_GRAPHS, 1), jnp.float32),
        scratch_shapes=[
            pltpu.VMEM((NUM_GRAPHS, EMB), jnp.float32),
            pltpu.VMEM((NUM_GRAPHS, 1), jnp.float32),
            pltpu.VMEM((N_SOLV, EMB), jnp.float32),
        ],
    )(srep, vrep, sb_f, vb_f, wo0, bo0, wo1, bo1, wo2, bo2, wl, bl)


def kernel(solute_x, solute_edge_index, solute_batch, solvent_x, solvent_batch,
           W_embed, b_embed, W_gcn, b_gcn, bn_gamma, bn_beta,
           Ws1, bs1, Ws2, bs2, Wo0, bo0, Wo1, bo1, Wo2, bo2, Wlast, blast):
    src = solute_edge_index[0]
    dst = solute_edge_index[1]
    pad = E_PAD - N_EDGES
    src_r = jnp.concatenate(
        [src, jnp.zeros((pad,), jnp.int32)]).reshape(NW, STEPS, EB)
    dst_r = jnp.concatenate(
        [dst, jnp.full((pad,), N_NODES, jnp.int32)]).reshape(NW, STEPS, EB)
    z = jnp.zeros((R_T, EMB), jnp.float32)

    h = _embed(solute_x, W_embed, b_embed.reshape(1, EMB))
    for l in range(NUM_LAYER):
        p = _edge_agg(h, src_r, dst_r, z)
        h = _layer(p, W_gcn[l], b_gcn[l].reshape(1, EMB),
                   bn_gamma[l].reshape(1, EMB), bn_beta[l].reshape(1, EMB),
                   relu=(l < NUM_LAYER - 1))

    vrep = _solvent(solvent_x, Ws1, bs1.reshape(1, EMB), Ws2, bs2.reshape(1, EMB))
    out = _interact(h, vrep,
                    solute_batch.astype(jnp.float32).reshape(N_NODES, 1),
                    solvent_batch.astype(jnp.float32).reshape(N_SOLV, 1),
                    Wo0, bo0.reshape(1, EMB),
                    Wo1, bo1.reshape(1, EMB // 2),
                    Wo2, bo2.reshape(1, EMB // 4),
                    Wlast, blast.reshape(1, 1))
    return out


# speedup vs baseline: 10.2574x; 1.2645x over previous
import functools

import jax
import jax.numpy as jnp
from jax import lax
from jax.experimental import pallas as pl
from jax.experimental.pallas import tpu as pltpu
import jax.experimental.pallas.tpu_sc as plsc

N_NODES = 10000
N_EDGES = 640000
N_SOLV = 2560
NUM_GRAPHS = 128
D_ATOM = 40
EMB = 64
NUM_LAYER = 3

NC = 2
NS = 16
NW = NC * NS
EB = 128
STEPS = 158
E_PAD = NW * STEPS * EB
N_PAD = 10240
R_T = N_PAD // NS

_HIGH = jax.lax.Precision.DEFAULT


NB = 4


def _edge_agg_body(h_hbm, src_hbm, dst_hbm, z_hbm, out_hbm,
                   src_v, dst_v, rows_v, agg_sh, gsem, ssem):
    c = lax.axis_index("c")
    s = lax.axis_index("s")
    wid = c * NS + s
    pltpu.sync_copy(z_hbm, agg_sh.at[pl.ds(s * R_T, R_T)])
    pltpu.sync_copy(src_hbm.at[wid], src_v)
    pltpu.sync_copy(dst_hbm.at[wid], dst_v)
    plsc.subcore_barrier()

    for b in range(2):
        pltpu.async_copy(h_hbm.at[src_v.at[b]], rows_v.at[b], gsem.at[b])

    def step(j, carry):
        buf = lax.rem(j, NB)
        pltpu.make_async_copy(h_hbm.at[src_v.at[j]], rows_v.at[buf],
                              gsem.at[buf]).wait()
        pltpu.async_copy(rows_v.at[buf], agg_sh.at[dst_v.at[j]],
                         ssem.at[buf], add=True)

        nbuf = lax.rem(j + 2, NB)

        @pl.when(jnp.logical_and(j >= 2, j + 2 < STEPS))
        def _drain():
            pltpu.make_async_copy(rows_v.at[nbuf],
                                  agg_sh.at[dst_v.at[j - 2]],
                                  ssem.at[nbuf]).wait()

        @pl.when(j + 2 < STEPS)
        def _refill():
            pltpu.async_copy(h_hbm.at[src_v.at[j + 2]], rows_v.at[nbuf],
                             gsem.at[nbuf])

        return carry

    lax.fori_loop(0, STEPS, step, 0)
    for jj in range(STEPS - 4, STEPS):
        pltpu.make_async_copy(rows_v.at[jj % NB], agg_sh.at[dst_v.at[jj]],
                              ssem.at[jj % NB]).wait()
    plsc.subcore_barrier()
    base = c * N_PAD + s * R_T
    pltpu.sync_copy(agg_sh.at[pl.ds(s * R_T, R_T)], out_hbm.at[pl.ds(base, R_T)])


def _edge_agg(h, src_r, dst_r, z):
    mesh = plsc.VectorSubcoreMesh(core_axis_name="c", subcore_axis_name="s",
                                  num_cores=NC, num_subcores=NS)
    return pl.kernel(
        _edge_agg_body,
        out_type=jax.ShapeDtypeStruct((NC * N_PAD, EMB), jnp.float32),
        mesh=mesh,
        scratch_types=[
            pltpu.VMEM((STEPS, EB), jnp.int32),
            pltpu.VMEM((STEPS, EB), jnp.int32),
            pltpu.VMEM((NB, EB, EMB), jnp.float32),
            pltpu.VMEM_SHARED((N_PAD, EMB), jnp.float32),
            pltpu.SemaphoreType.DMA((NB,)),
            pltpu.SemaphoreType.DMA((NB,)),
        ],
        compiler_params=pltpu.CompilerParams(use_tc_tiling_on_sc=False),
    )(h, src_r, dst_r, z)


def _embed_body(x_ref, w_ref, b_ref, o_ref):
    o_ref[...] = jax.nn.relu(
        jnp.dot(x_ref[...], w_ref[...], precision=_HIGH,
                preferred_element_type=jnp.float32) + b_ref[...])


def _embed(x, w, b):
    return pl.pallas_call(
        _embed_body,
        out_shape=jax.ShapeDtypeStruct((N_NODES, EMB), jnp.float32),
    )(x, w, b)


def _layer_body(p0_ref, p1_ref, w_ref, b_ref, g_ref, be_ref, o_ref, *, relu):
    agg = p0_ref[...] + p1_ref[...]
    hl = jnp.dot(agg, w_ref[...], precision=_HIGH,
                 preferred_element_type=jnp.float32) + b_ref[...]
    mean = jnp.mean(hl, axis=0, keepdims=True)
    cen = hl - mean
    var = jnp.mean(cen * cen, axis=0, keepdims=True)
    hl = cen * jax.lax.rsqrt(var + 1e-5) * g_ref[...] + be_ref[...]
    if relu:
        hl = jax.nn.relu(hl)
    o_ref[...] = hl


def _layer(p, w, b, g, be, relu):
    p0 = p[:N_NODES]
    p1 = p[N_PAD:N_PAD + N_NODES]
    return pl.pallas_call(
        functools.partial(_layer_body, relu=relu),
        out_shape=jax.ShapeDtypeStruct((N_NODES, EMB), jnp.float32),
    )(p0, p1, w, b, g, be)


def _solvent_body(x_ref, w1_ref, b1_ref, w2_ref, b2_ref, o_ref):
    h = jax.nn.relu(
        jnp.dot(x_ref[...], w1_ref[...], precision=_HIGH,
                preferred_element_type=jnp.float32) + b1_ref[...])
    o_ref[...] = jnp.dot(h, w2_ref[...], precision=_HIGH,
                         preferred_element_type=jnp.float32) + b2_ref[...]


def _solvent(x, w1, b1, w2, b2):
    return pl.pallas_call(
        _solvent_body,
        out_shape=jax.ShapeDtypeStruct((N_SOLV, EMB), jnp.float32),
    )(x, w1, b1, w2, b2)


SB = 400
NSB = N_NODES // SB


def _interact_body(srep_ref, vrep_ref, sb_ref, vb_ref,
                   wo0_ref, bo0_ref, wo1_ref, bo1_ref, wo2_ref, bo2_ref,
                   wl_ref, bl_ref, o_ref,
                   sfsum, scnt, vprime):
    i = pl.program_id(0)
    vrep = vrep_ref[...]

    @pl.when(i == 0)
    def _init():
        sfsum[...] = jnp.zeros_like(sfsum)
        scnt[...] = jnp.zeros_like(scnt)
        vprime[...] = jnp.zeros_like(vprime)

    rep = srep_ref[...]
    sb = sb_ref[...]
    mask = (sb == vb_ref[...].reshape(1, N_SOLV)).astype(jnp.float32)
    imap = lax.dot_general(rep, vrep, (((1,), (1,)), ((), ())),
                           precision=_HIGH,
                           preferred_element_type=jnp.float32) * mask
    sp = jnp.dot(imap, vrep, precision=_HIGH,
                 preferred_element_type=jnp.float32)
    sf = rep + sp
    gids = lax.broadcasted_iota(jnp.int32, (NUM_GRAPHS, 1), 0).astype(jnp.float32)
    p_s = (gids == sb.reshape(1, SB)).astype(jnp.float32)
    sfsum[...] += jnp.dot(p_s, sf, precision=_HIGH,
                          preferred_element_type=jnp.float32)
    scnt[...] += jnp.sum(p_s, axis=1, keepdims=True)
    vprime[...] += lax.dot_general(imap, rep, (((0,), (0,)), ((), ())),
                                   precision=_HIGH,
                                   preferred_element_type=jnp.float32)

    @pl.when(i == NSB - 1)
    def _final():
        vf = vrep + vprime[...]
        p_v = (gids == vb_ref[...].reshape(1, N_SOLV)).astype(jnp.float32)
        vsum = jnp.dot(p_v, vf, precision=_HIGH,
                       preferred_element_type=jnp.float32)
        vcnt = jnp.sum(p_v, axis=1, keepdims=True)
        pooled_s = sfsum[...] / jnp.maximum(scnt[...], 1.0)
        pooled_v = vsum / jnp.maximum(vcnt, 1.0)
        final = jnp.concatenate([pooled_s, pooled_v], axis=1)
        h1 = jax.nn.relu(jnp.dot(final, wo0_ref[...], precision=_HIGH,
                                 preferred_element_type=jnp.float32) + bo0_ref[...])
        h2 = jax.nn.relu(jnp.dot(h1, wo1_ref[...], precision=_HIGH,
                                 preferred_element_type=jnp.float32) + bo1_ref[...])
        h3 = jax.nn.relu(jnp.dot(h2, wo2_ref[...], precision=_HIGH,
                                 preferred_element_type=jnp.float32) + bo2_ref[...])
        o_ref[...] = jnp.dot(h3, wl_ref[...], precision=_HIGH,
                             preferred_element_type=jnp.float32) + bl_ref[...]


def _interact(srep, vrep, sb_f, vb_f, wo0, bo0, wo1, bo1, wo2, bo2, wl, bl):
    const = lambda i: (0, 0)
    return pl.pallas_call(
        _interact_body,
        grid=(NSB,),
        in_specs=[
            pl.BlockSpec((SB, EMB), lambda i: (i, 0)),
            pl.BlockSpec((N_SOLV, EMB), const),
            pl.BlockSpec((SB, 1), lambda i: (i, 0)),
            pl.BlockSpec((N_SOLV, 1), const),
            pl.BlockSpec((2 * EMB, EMB), const),
            pl.BlockSpec((1, EMB), const),
            pl.BlockSpec((EMB, EMB // 2), const),
            pl.BlockSpec((1, EMB // 2), const),
            pl.BlockSpec((EMB // 2, EMB // 4), const),
            pl.BlockSpec((1, EMB // 4), const),
            pl.BlockSpec((EMB // 4, 1), const),
            pl.BlockSpec((1, 1), const),
        ],
        out_specs=pl.BlockSpec((NUM_GRAPHS, 1), const),
        out_shape=jax.ShapeDtypeStruct((NUM_GRAPHS, 1), jnp.float32),
        scratch_shapes=[
            pltpu.VMEM((NUM_GRAPHS, EMB), jnp.float32),
            pltpu.VMEM((NUM_GRAPHS, 1), jnp.float32),
            pltpu.VMEM((N_SOLV, EMB), jnp.float32),
        ],
    )(srep, vrep, sb_f, vb_f, wo0, bo0, wo1, bo1, wo2, bo2, wl, bl)


def kernel(solute_x, solute_edge_index, solute_batch, solvent_x, solvent_batch,
           W_embed, b_embed, W_gcn, b_gcn, bn_gamma, bn_beta,
           Ws1, bs1, Ws2, bs2, Wo0, bo0, Wo1, bo1, Wo2, bo2, Wlast, blast):
    src = solute_edge_index[0]
    dst = solute_edge_index[1]
    pad = E_PAD - N_EDGES
    src_r = jnp.concatenate(
        [src, jnp.zeros((pad,), jnp.int32)]).reshape(NW, STEPS, EB)
    dst_r = jnp.concatenate(
        [dst, jnp.full((pad,), N_NODES, jnp.int32)]).reshape(NW, STEPS, EB)
    z = jnp.zeros((R_T, EMB), jnp.float32)

    h = _embed(solute_x, W_embed, b_embed.reshape(1, EMB))
    for l in range(NUM_LAYER):
        p = _edge_agg(h, src_r, dst_r, z)
        h = _layer(p, W_gcn[l], b_gcn[l].reshape(1, EMB),
                   bn_gamma[l].reshape(1, EMB), bn_beta[l].reshape(1, EMB),
                   relu=(l < NUM_LAYER - 1))

    vrep = _solvent(solvent_x, Ws1, bs1.reshape(1, EMB), Ws2, bs2.reshape(1, EMB))
    out = _interact(h, vrep,
                    solute_batch.astype(jnp.float32).reshape(N_NODES, 1),
                    solvent_batch.astype(jnp.float32).reshape(N_SOLV, 1),
                    Wo0, bo0.reshape(1, EMB),
                    Wo1, bo1.reshape(1, EMB // 2),
                    Wo2, bo2.reshape(1, EMB // 4),
                    Wlast, blast.reshape(1, 1))
    return out
